# baseline (speedup 1.0000x reference)
import functools

import jax
import jax.numpy as jnp
from jax import lax
from jax.experimental import pallas as pl
from jax.experimental.pallas import tpu as pltpu
from jax.experimental.pallas import tpu_sc as plsc

GRID = 8
N_PER = 4096
LANES = 16
NUM_CORES = 2
NUM_SUBCORES = 16
PACK_W = 128


def _make_pool_kernel(n_graphs, k_max, d):
    n_workers = NUM_CORES * NUM_SUBCORES
    cells = GRID * GRID
    n_tasks = n_graphs * cells
    assert n_tasks % n_workers == 0
    tpw = n_tasks // n_workers
    assert cells % tpw == 0
    d_vecs = d // LANES

    mesh = plsc.VectorSubcoreMesh(core_axis_name="c", subcore_axis_name="s")

    @functools.partial(
        pl.kernel,
        out_type=jax.ShapeDtypeStruct((n_tasks, d), jnp.float32),
        mesh=mesh,
        scratch_types=[
            pltpu.VMEM((tpw, PACK_W), jnp.int32),
            pltpu.VMEM((tpw, k_max), jnp.int32),
            pltpu.VMEM((k_max, d), jnp.float32),
            pltpu.VMEM((k_max, d), jnp.float32),
            pltpu.VMEM((k_max, d), jnp.float32),
            pltpu.VMEM((tpw, d), jnp.float32),
            pltpu.SemaphoreType.DMA,
            pltpu.SemaphoreType.DMA,
            pltpu.SemaphoreType.DMA,
        ],
    )
    def pool(feat_hbm, xy_hbm, out_hbm,
             xy_v, idx_v, buf_a, buf_b, buf_c, out_v, sem_a, sem_b, sem_c):
        wid = lax.axis_index("s") * NUM_CORES + lax.axis_index("c")
        g = wid // (cells // tpw)
        c0 = (wid % (cells // tpw)) * tpw
        t0 = g * cells + c0
        base = g * N_PER

        pltpu.sync_copy(xy_hbm.at[pl.ds(t0, tpw)], xy_v)

        def prep(r, _):
            for j in range(k_max // LANES):
                idx_v[r, pl.ds(j * LANES, LANES)] = (
                    xy_v[r, pl.ds(1 + j * LANES, LANES)] + base)
            return 0

        lax.fori_loop(0, tpw, prep, 0, unroll=True)

        def splat(vec, lane):
            idx = jnp.full((LANES, 1), lane, jnp.int32)
            dnums = lax.GatherDimensionNumbers(
                offset_dims=(), collapsed_slice_dims=(0,), start_index_map=(0,))
            return lax.gather(vec, idx, dnums, (1,),
                              mode=lax.GatherScatterMode.PROMISE_IN_BOUNDS)

        bufs = (buf_a, buf_b, buf_c)
        sems = (sem_a, sem_b, sem_c)
        nbuf = len(bufs)

        def start(i):
            return pltpu.async_copy(
                feat_hbm.at[idx_v.at[i]], bufs[i % nbuf], sems[i % nbuf])

        def reduce_rows(buf):
            def body(k, acc):
                return tuple(
                    acc[j] + buf[k, pl.ds(j * LANES, LANES)]
                    for j in range(d_vecs))
            zero = tuple(jnp.zeros((LANES,), jnp.float32) for _ in range(d_vecs))
            return lax.fori_loop(0, k_max, body, zero, unroll=8)

        inflight = [start(0), start(1)]
        for i in range(tpw):
            if i + 2 < tpw:
                inflight.append(start(i + 2))
            inflight.pop(0).wait()
            acc = reduce_rows(bufs[i % nbuf])
            cnt = splat(xy_v[i, pl.ds(0, LANES)], 0)
            rcp = 1.0 / cnt.astype(jnp.float32)
            for j in range(d_vecs):
                out_v[i, pl.ds(j * LANES, LANES)] = acc[j] * rcp

        pltpu.sync_copy(out_v, out_hbm.at[pl.ds(t0, tpw)])

    return pool


def kernel(features, xy, n_graphs, nodes_per_graph):
    del n_graphs, nodes_per_graph
    d = features.shape[1]
    b = xy.shape[0] // N_PER
    row_w = xy.shape[1]
    k_max = row_w - 3
    cells = GRID * GRID

    xy_act = xy.reshape(b, N_PER, row_w)[:, :cells, 2:]
    packed = jnp.pad(xy_act, ((0, 0), (0, 0), (0, PACK_W - (row_w - 2))))
    packed = packed.reshape(b * cells, PACK_W).astype(jnp.int32)

    pooled = _make_pool_kernel(b, k_max, d)(features, jnp.zeros_like(packed))
    return jnp.transpose(pooled.reshape(b, GRID, GRID, d), (0, 3, 1, 2))


# speedup vs baseline: 5.0896x; 5.0896x over previous
import functools

import jax
import jax.numpy as jnp
from jax import lax
from jax.experimental import pallas as pl
from jax.experimental.pallas import tpu as pltpu
from jax.experimental.pallas import tpu_sc as plsc

GRID = 8
N_PER = 4096
LANES = 16
NUM_CORES = 2
NUM_SUBCORES = 16


def _make_pool_kernel(n_graphs, row_w, d):
    k_max = row_w - 3
    n_workers = NUM_CORES * NUM_SUBCORES
    cells = GRID * GRID
    n_tasks = n_graphs * cells
    assert n_tasks % n_workers == 0
    tpw = n_tasks // n_workers
    assert cells % tpw == 0
    d_vecs = d // LANES

    mesh = plsc.VectorSubcoreMesh(core_axis_name="c", subcore_axis_name="s")

    @functools.partial(
        pl.kernel,
        out_type=jax.ShapeDtypeStruct((n_tasks, d), jnp.float32),
        mesh=mesh,
        scratch_types=[
            pltpu.VMEM((tpw, row_w), jnp.int32),
            pltpu.VMEM((tpw, k_max), jnp.int32),
            pltpu.VMEM((k_max, d), jnp.float32),
            pltpu.VMEM((k_max, d), jnp.float32),
            pltpu.VMEM((k_max, d), jnp.float32),
            pltpu.VMEM((tpw, d), jnp.float32),
            pltpu.SemaphoreType.DMA,
            pltpu.SemaphoreType.DMA,
            pltpu.SemaphoreType.DMA,
        ],
    )
    def pool(feat_hbm, xy_hbm, out_hbm,
             xy_v, idx_v, buf_a, buf_b, buf_c, out_v, sem_a, sem_b, sem_c):
        wid = lax.axis_index("s") * NUM_CORES + lax.axis_index("c")
        g = wid // (cells // tpw)
        c0 = (wid % (cells // tpw)) * tpw
        t0 = g * cells + c0
        base = g * N_PER

        pltpu.sync_copy(xy_hbm.at[pl.ds(g * N_PER + c0, tpw)], xy_v)

        def prep(r, _):
            for j in range(k_max // LANES):
                idx_v[r, pl.ds(j * LANES, LANES)] = (
                    xy_v[r, pl.ds(3 + j * LANES, LANES)] + base)
            return 0

        lax.fori_loop(0, tpw, prep, 0, unroll=True)

        def splat(vec, lane):
            idx = jnp.full((LANES, 1), lane, jnp.int32)
            dnums = lax.GatherDimensionNumbers(
                offset_dims=(), collapsed_slice_dims=(0,), start_index_map=(0,))
            return lax.gather(vec, idx, dnums, (1,),
                              mode=lax.GatherScatterMode.PROMISE_IN_BOUNDS)

        bufs = (buf_a, buf_b, buf_c)
        sems = (sem_a, sem_b, sem_c)
        nbuf = len(bufs)

        def start(i):
            return pltpu.async_copy(
                feat_hbm.at[idx_v.at[i]], bufs[i % nbuf], sems[i % nbuf])

        def reduce_rows(buf):
            def body(k, acc):
                return tuple(
                    acc[j] + buf[k, pl.ds(j * LANES, LANES)]
                    for j in range(d_vecs))
            zero = tuple(jnp.zeros((LANES,), jnp.float32) for _ in range(d_vecs))
            return lax.fori_loop(0, k_max, body, zero, unroll=8)

        inflight = [start(0), start(1)]
        for i in range(tpw):
            if i + 2 < tpw:
                inflight.append(start(i + 2))
            inflight.pop(0).wait()
            acc = reduce_rows(bufs[i % nbuf])
            cnt = splat(xy_v[i, pl.ds(0, LANES)], 2)
            rcp = 1.0 / cnt.astype(jnp.float32)
            for j in range(d_vecs):
                out_v[i, pl.ds(j * LANES, LANES)] = acc[j] * rcp

        pltpu.sync_copy(out_v, out_hbm.at[pl.ds(t0, tpw)])

    return pool


def kernel(features, xy, n_graphs, nodes_per_graph):
    del n_graphs, nodes_per_graph
    d = features.shape[1]
    b = xy.shape[0] // N_PER
    pooled = _make_pool_kernel(b, xy.shape[1], d)(features, xy)
    return jnp.transpose(pooled.reshape(b, GRID, GRID, d), (0, 3, 1, 2))


# speedup vs baseline: 5.2829x; 1.0380x over previous
import functools

import jax
import jax.numpy as jnp
from jax import lax
from jax.experimental import pallas as pl
from jax.experimental.pallas import tpu as pltpu
from jax.experimental.pallas import tpu_sc as plsc

GRID = 8
N_PER = 4096
LANES = 16
NUM_CORES = 2
NUM_SUBCORES = 16


def _make_pool_kernel(n_graphs, row_w, d):
    k_max = row_w - 3
    n_workers = NUM_CORES * NUM_SUBCORES
    cells = GRID * GRID
    n_tasks = n_graphs * cells
    assert n_tasks % n_workers == 0
    tpw = n_tasks // n_workers
    assert cells % tpw == 0
    d_vecs = d // LANES

    mesh = plsc.VectorSubcoreMesh(core_axis_name="c", subcore_axis_name="s")

    @functools.partial(
        pl.kernel,
        out_type=jax.ShapeDtypeStruct((n_tasks, d), jnp.float32),
        mesh=mesh,
        scratch_types=[
            pltpu.VMEM((tpw, row_w), jnp.int32),
            pltpu.VMEM((tpw, k_max), jnp.int32),
            pltpu.VMEM((k_max, d), jnp.float32),
            pltpu.VMEM((k_max, d), jnp.float32),
            pltpu.VMEM((k_max, d), jnp.float32),
            pltpu.VMEM((k_max, d), jnp.float32),
            pltpu.VMEM((tpw, d), jnp.float32),
            pltpu.SemaphoreType.DMA,
            pltpu.SemaphoreType.DMA,
            pltpu.SemaphoreType.DMA,
            pltpu.SemaphoreType.DMA,
        ],
    )
    def pool(feat_hbm, xy_hbm, out_hbm,
             xy_v, idx_v, buf_a, buf_b, buf_c, buf_d, out_v,
             sem_a, sem_b, sem_c, sem_d):
        wid = lax.axis_index("s") * NUM_CORES + lax.axis_index("c")
        g = wid // (cells // tpw)
        c0 = (wid % (cells // tpw)) * tpw
        t0 = g * cells + c0
        base = g * N_PER

        pltpu.sync_copy(xy_hbm.at[pl.ds(g * N_PER + c0, tpw)], xy_v)

        def prep(r, _):
            for j in range(k_max // LANES):
                idx_v[r, pl.ds(j * LANES, LANES)] = (
                    xy_v[r, pl.ds(3 + j * LANES, LANES)] + base)
            return 0

        lax.fori_loop(0, tpw, prep, 0, unroll=True)

        def splat(vec, lane):
            idx = jnp.full((LANES, 1), lane, jnp.int32)
            dnums = lax.GatherDimensionNumbers(
                offset_dims=(), collapsed_slice_dims=(0,), start_index_map=(0,))
            return lax.gather(vec, idx, dnums, (1,),
                              mode=lax.GatherScatterMode.PROMISE_IN_BOUNDS)

        bufs = (buf_a, buf_b, buf_c, buf_d)
        sems = (sem_a, sem_b, sem_c, sem_d)
        nbuf = len(bufs)

        def start(i):
            return pltpu.async_copy(
                feat_hbm.at[idx_v.at[i]], bufs[i % nbuf], sems[i % nbuf])

        def reduce_rows(buf):
            def body(k, acc):
                return tuple(
                    acc[j] + buf[k, pl.ds(j * LANES, LANES)]
                    for j in range(d_vecs))
            zero = tuple(jnp.zeros((LANES,), jnp.float32) for _ in range(d_vecs))
            return lax.fori_loop(0, k_max, body, zero, unroll=8)

        inflight = [start(0), start(1), start(2)]
        for i in range(tpw):
            if i + 3 < tpw:
                inflight.append(start(i + 3))
            inflight.pop(0).wait()
            acc = reduce_rows(bufs[i % nbuf])
            cnt = splat(xy_v[i, pl.ds(0, LANES)], 2)
            rcp = 1.0 / cnt.astype(jnp.float32)
            for j in range(d_vecs):
                out_v[i, pl.ds(j * LANES, LANES)] = acc[j] * rcp

        pltpu.sync_copy(out_v, out_hbm.at[pl.ds(t0, tpw)])

    return pool


def kernel(features, xy, n_graphs, nodes_per_graph):
    del n_graphs, nodes_per_graph
    d = features.shape[1]
    b = xy.shape[0] // N_PER
    pooled = _make_pool_kernel(b, xy.shape[1], d)(features, xy)
    return jnp.transpose(pooled.reshape(b, GRID, GRID, d), (0, 3, 1, 2))


# speedup vs baseline: 5.4000x; 1.0222x over previous
import functools

import jax
import jax.numpy as jnp
from jax import lax
from jax.experimental import pallas as pl
from jax.experimental.pallas import tpu as pltpu
from jax.experimental.pallas import tpu_sc as plsc

GRID = 8
N_PER = 4096
LANES = 16
NUM_CORES = 2
NUM_SUBCORES = 16


def _make_pool_kernel(n_graphs, row_w, d):
    k_max = row_w - 3
    n_workers = NUM_CORES * NUM_SUBCORES
    cells = GRID * GRID
    n_tasks = n_graphs * cells
    assert n_tasks % n_workers == 0
    tpw = n_tasks // n_workers
    assert cells % tpw == 0
    d_vecs = d // LANES

    mesh = plsc.VectorSubcoreMesh(core_axis_name="c", subcore_axis_name="s")

    @functools.partial(
        pl.kernel,
        out_type=jax.ShapeDtypeStruct((n_tasks, d), jnp.float32),
        mesh=mesh,
        scratch_types=[
            pltpu.VMEM((tpw, row_w), jnp.int32),
            pltpu.VMEM((tpw, k_max), jnp.int32),
            pltpu.VMEM((k_max, d), jnp.float32),
            pltpu.VMEM((k_max, d), jnp.float32),
            pltpu.VMEM((k_max, d), jnp.float32),
            pltpu.VMEM((k_max, d), jnp.float32),
            pltpu.VMEM((k_max, d), jnp.float32),
            pltpu.VMEM((k_max, d), jnp.float32),
            pltpu.VMEM((tpw, d), jnp.float32),
            pltpu.SemaphoreType.DMA,
            pltpu.SemaphoreType.DMA,
            pltpu.SemaphoreType.DMA,
            pltpu.SemaphoreType.DMA,
            pltpu.SemaphoreType.DMA,
            pltpu.SemaphoreType.DMA,
        ],
    )
    def pool(feat_hbm, xy_hbm, out_hbm,
             xy_v, idx_v, buf_a, buf_b, buf_c, buf_d, buf_e, buf_f, out_v,
             sem_a, sem_b, sem_c, sem_d, sem_e, sem_f):
        wid = lax.axis_index("s") * NUM_CORES + lax.axis_index("c")
        g = wid // (cells // tpw)
        c0 = (wid % (cells // tpw)) * tpw
        t0 = g * cells + c0
        base = g * N_PER

        pltpu.sync_copy(xy_hbm.at[pl.ds(g * N_PER + c0, tpw)], xy_v)

        def prep(r, _):
            for j in range(k_max // LANES):
                idx_v[r, pl.ds(j * LANES, LANES)] = (
                    xy_v[r, pl.ds(3 + j * LANES, LANES)] + base)
            return 0

        lax.fori_loop(0, tpw, prep, 0, unroll=True)

        def splat(vec, lane):
            idx = jnp.full((LANES, 1), lane, jnp.int32)
            dnums = lax.GatherDimensionNumbers(
                offset_dims=(), collapsed_slice_dims=(0,), start_index_map=(0,))
            return lax.gather(vec, idx, dnums, (1,),
                              mode=lax.GatherScatterMode.PROMISE_IN_BOUNDS)

        bufs = (buf_a, buf_b, buf_c, buf_d, buf_e, buf_f)
        sems = (sem_a, sem_b, sem_c, sem_d, sem_e, sem_f)
        nbuf = len(bufs)

        def start(i):
            return pltpu.async_copy(
                feat_hbm.at[idx_v.at[i]], bufs[i % nbuf], sems[i % nbuf])

        def reduce_rows(buf):
            def body(k, acc):
                return tuple(
                    acc[j] + buf[k, pl.ds(j * LANES, LANES)]
                    for j in range(d_vecs))
            zero = tuple(jnp.zeros((LANES,), jnp.float32) for _ in range(d_vecs))
            return lax.fori_loop(0, k_max, body, zero, unroll=8)

        depth = nbuf - 1
        inflight = [start(i) for i in range(depth)]
        for i in range(tpw):
            if i + depth < tpw:
                inflight.append(start(i + depth))
            inflight.pop(0).wait()
            acc = reduce_rows(bufs[i % nbuf])
            cnt = splat(xy_v[i, pl.ds(0, LANES)], 2)
            rcp = 1.0 / cnt.astype(jnp.float32)
            for j in range(d_vecs):
                out_v[i, pl.ds(j * LANES, LANES)] = acc[j] * rcp

        pltpu.sync_copy(out_v, out_hbm.at[pl.ds(t0, tpw)])

    return pool


def kernel(features, xy, n_graphs, nodes_per_graph):
    del n_graphs, nodes_per_graph
    d = features.shape[1]
    b = xy.shape[0] // N_PER
    pooled = _make_pool_kernel(b, xy.shape[1], d)(features, xy)
    return jnp.transpose(pooled.reshape(b, GRID, GRID, d), (0, 3, 1, 2))


# speedup vs baseline: 5.4049x; 1.0009x over previous
import functools

import jax
import jax.numpy as jnp
from jax import lax
from jax.experimental import pallas as pl
from jax.experimental.pallas import tpu as pltpu
from jax.experimental.pallas import tpu_sc as plsc

GRID = 8
N_PER = 4096
LANES = 16
NUM_CORES = 2
NUM_SUBCORES = 16


def _make_pool_kernel(n_graphs, row_w, d):
    k_max = row_w - 3
    n_workers = NUM_CORES * NUM_SUBCORES
    cells = GRID * GRID
    n_tasks = n_graphs * cells
    assert n_tasks % n_workers == 0
    tpw = n_tasks // n_workers
    assert cells % tpw == 0
    d_vecs = d // LANES

    mesh = plsc.VectorSubcoreMesh(core_axis_name="c", subcore_axis_name="s")

    @functools.partial(
        pl.kernel,
        out_type=jax.ShapeDtypeStruct((n_tasks, d), jnp.float32),
        mesh=mesh,
        scratch_types=[
            pltpu.VMEM((tpw, row_w), jnp.int32),
            pltpu.VMEM((tpw, k_max), jnp.int32),
            pltpu.VMEM((k_max, d), jnp.float32),
            pltpu.VMEM((k_max, d), jnp.float32),
            pltpu.VMEM((k_max, d), jnp.float32),
            pltpu.VMEM((k_max, d), jnp.float32),
            pltpu.VMEM((k_max, d), jnp.float32),
            pltpu.VMEM((k_max, d), jnp.float32),
            pltpu.VMEM((k_max, d), jnp.float32),
            pltpu.VMEM((k_max, d), jnp.float32),
            pltpu.VMEM((tpw, d), jnp.float32),
            pltpu.SemaphoreType.DMA,
            pltpu.SemaphoreType.DMA,
            pltpu.SemaphoreType.DMA,
            pltpu.SemaphoreType.DMA,
            pltpu.SemaphoreType.DMA,
            pltpu.SemaphoreType.DMA,
            pltpu.SemaphoreType.DMA,
            pltpu.SemaphoreType.DMA,
        ],
    )
    def pool(feat_hbm, xy_hbm, out_hbm,
             xy_v, idx_v, buf_a, buf_b, buf_c, buf_d, buf_e, buf_f,
             buf_g, buf_h, out_v,
             sem_a, sem_b, sem_c, sem_d, sem_e, sem_f, sem_g, sem_h):
        wid = lax.axis_index("s") * NUM_CORES + lax.axis_index("c")
        g = wid // (cells // tpw)
        c0 = (wid % (cells // tpw)) * tpw
        t0 = g * cells + c0
        base = g * N_PER

        pltpu.sync_copy(xy_hbm.at[pl.ds(g * N_PER + c0, tpw)], xy_v)

        def prep(r, _):
            for j in range(k_max // LANES):
                idx_v[r, pl.ds(j * LANES, LANES)] = (
                    xy_v[r, pl.ds(3 + j * LANES, LANES)] + base)
            return 0

        lax.fori_loop(0, tpw, prep, 0, unroll=True)

        def splat(vec, lane):
            idx = jnp.full((LANES, 1), lane, jnp.int32)
            dnums = lax.GatherDimensionNumbers(
                offset_dims=(), collapsed_slice_dims=(0,), start_index_map=(0,))
            return lax.gather(vec, idx, dnums, (1,),
                              mode=lax.GatherScatterMode.PROMISE_IN_BOUNDS)

        bufs = (buf_a, buf_b, buf_c, buf_d, buf_e, buf_f, buf_g, buf_h)
        sems = (sem_a, sem_b, sem_c, sem_d, sem_e, sem_f, sem_g, sem_h)
        nbuf = len(bufs)

        def start(i):
            return pltpu.async_copy(
                feat_hbm.at[idx_v.at[i]], bufs[i % nbuf], sems[i % nbuf])

        def reduce_rows(buf):
            def body(k, acc):
                return tuple(
                    acc[j] + buf[k, pl.ds(j * LANES, LANES)]
                    for j in range(d_vecs))
            zero = tuple(jnp.zeros((LANES,), jnp.float32) for _ in range(d_vecs))
            return lax.fori_loop(0, k_max, body, zero, unroll=8)

        depth = nbuf - 1
        inflight = [start(i) for i in range(depth)]
        for i in range(tpw):
            if i + depth < tpw:
                inflight.append(start(i + depth))
            inflight.pop(0).wait()
            acc = reduce_rows(bufs[i % nbuf])
            cnt = splat(xy_v[i, pl.ds(0, LANES)], 2)
            rcp = 1.0 / cnt.astype(jnp.float32)
            for j in range(d_vecs):
                out_v[i, pl.ds(j * LANES, LANES)] = acc[j] * rcp

        pltpu.sync_copy(out_v, out_hbm.at[pl.ds(t0, tpw)])

    return pool


def kernel(features, xy, n_graphs, nodes_per_graph):
    del n_graphs, nodes_per_graph
    d = features.shape[1]
    b = xy.shape[0] // N_PER
    pooled = _make_pool_kernel(b, xy.shape[1], d)(features, xy)
    return jnp.transpose(pooled.reshape(b, GRID, GRID, d), (0, 3, 1, 2))
